# baseline (speedup 1.0000x reference)
import functools

import jax
import jax.numpy as jnp
from jax import lax
from jax.experimental import pallas as pl
from jax.experimental.pallas import tpu as pltpu
from jax.experimental.pallas import tpu_sc as plsc

E = 64
HIDDEN = 1024
FFN = 2048
T = 8192
TM = 128
NP = T // TM + E
P = NP * TM


_NW = 32


def _make_row_gather(B, D, dtype, CH):
    bpw = B // _NW
    nch = bpw // CH
    mesh = plsc.VectorSubcoreMesh(core_axis_name="c", subcore_axis_name="s")

    @functools.partial(
        pl.kernel, mesh=mesh,
        out_type=jax.ShapeDtypeStruct((B, D), dtype),
        scratch_types=[
            pltpu.VMEM((nch, CH), jnp.int32),
            pltpu.VMEM((2, CH, D), dtype),
            pltpu.SemaphoreType.DMA,
            pltpu.SemaphoreType.DMA,
        ],
    )
    def gather_k(table_hbm, idx_hbm, out_hbm, idx_v, rows_v, sem0, sem1):
        wid = lax.axis_index("s") * 2 + lax.axis_index("c")
        base = wid * bpw
        pltpu.sync_copy(idx_hbm.at[pl.ds(wid * nch, nch)], idx_v)
        sems = [sem0, sem1]
        cps = [None, None]
        for c in range(nch):
            b = c % 2
            cps[b] = pltpu.async_copy(
                table_hbm.at[idx_v.at[c]], rows_v.at[b], sems[b])
            if c > 0:
                pb = (c - 1) % 2
                cps[pb].wait()
                pltpu.sync_copy(rows_v.at[pb],
                                out_hbm.at[pl.ds(base + (c - 1) * CH, CH)])
        lb = (nch - 1) % 2
        cps[lb].wait()
        pltpu.sync_copy(rows_v.at[lb],
                        out_hbm.at[pl.ds(base + (nch - 1) * CH, CH)])

    return gather_k


_CHUNK = T // _NW

_SC_MESH = dict(core_axis_name="c", subcore_axis_name="s")


def _sc_wid():
    return lax.axis_index("s") * 2 + lax.axis_index("c")


@functools.partial(
    pl.kernel, mesh=plsc.VectorSubcoreMesh(**_SC_MESH),
    out_type=jax.ShapeDtypeStruct((_NW, E), jnp.int32),
    scratch_types=[
        pltpu.VMEM((_CHUNK,), jnp.int32),
        pltpu.VMEM((E,), jnp.int32),
        pltpu.SMEM((E,), jnp.int32),
    ],
)
def _sc_histogram(top_hbm, hist_hbm, keys_v, hist_v, cur_s):
    wid = _sc_wid()
    pltpu.sync_copy(top_hbm.at[pl.ds(wid * _CHUNK, _CHUNK)], keys_v)
    for e in range(E):
        cur_s[e] = 0
    for v in range(_CHUNK // 16):
        kvec = keys_v[pl.ds(v * 16, 16)]
        for l in range(16):
            e = kvec[l]
            cur_s[e] = cur_s[e] + 1
    lanes = lax.broadcasted_iota(jnp.int32, (16,), 0)
    for g in range(E // 16):
        vec = jnp.zeros((16,), jnp.int32)
        for l in range(16):
            vec = jnp.where(lanes == l, cur_s[g * 16 + l], vec)
        hist_v[pl.ds(g * 16, 16)] = vec
    pltpu.sync_copy(hist_v, hist_hbm.at[wid])


@functools.partial(
    pl.kernel, mesh=plsc.VectorSubcoreMesh(**_SC_MESH),
    out_type=(
        jax.ShapeDtypeStruct((P, HIDDEN), jnp.float32),
        jax.ShapeDtypeStruct((P, 128), jnp.float32),
        jax.ShapeDtypeStruct((T // 32, 32), jnp.int32),
    ),
    scratch_types=[
        pltpu.VMEM((_CHUNK,), jnp.int32),
        pltpu.VMEM((8, 32), jnp.int32),
        pltpu.VMEM((2, 32, HIDDEN), jnp.float32),
        pltpu.VMEM((2, 32, 128), jnp.float32),
        pltpu.VMEM((E,), jnp.int32),
        pltpu.SMEM((E,), jnp.int32),
        pltpu.SemaphoreType.DMA,
        pltpu.SemaphoreType.DMA,
        pltpu.SemaphoreType.DMA,
        pltpu.SemaphoreType.DMA,
        pltpu.SemaphoreType.DMA,
        pltpu.SemaphoreType.DMA,
        pltpu.SemaphoreType.DMA,
        pltpu.SemaphoreType.DMA,
    ],
)
def _sc_place_scatter(top_hbm, start_hbm, hs_hbm, ew128_hbm,
                      xs_hbm, ews_hbm, inv_hbm,
                      keys_v, dest2_v, rows_v, ew_v, start_v, cur_s,
                      si0, si1, so0, so1, ei0, ei1, eo0, eo1):
    wid = _sc_wid()
    base = wid * _CHUNK
    pltpu.sync_copy(top_hbm.at[pl.ds(base, _CHUNK)], keys_v)
    pltpu.sync_copy(start_hbm.at[wid], start_v)
    for g in range(E // 16):
        svec = start_v[pl.ds(g * 16, 16)]
        for l in range(16):
            cur_s[g * 16 + l] = svec[l]
    lanes = lax.broadcasted_iota(jnp.int32, (16,), 0)
    for v in range(_CHUNK // 16):
        kvec = keys_v[pl.ds(v * 16, 16)]
        vec = jnp.zeros((16,), jnp.int32)
        for l in range(16):
            e = kvec[l]
            p = cur_s[e]
            cur_s[e] = p + 1
            vec = jnp.where(lanes == l, p, vec)
        dest2_v[v // 2, pl.ds((v % 2) * 16, 16)] = vec
    pltpu.sync_copy(dest2_v, inv_hbm.at[pl.ds(wid * 8, 8)])
    s_in = [(si0, ei0), (si1, ei1)]
    s_out = [(so0, eo0), (so1, eo1)]
    cp_in = [None, None]
    cp_out = [None, None]
    for c in range(8):
        b = c % 2
        if cp_out[b] is not None:
            for cp in cp_out[b]:
                cp.wait()
        cp_in[b] = (
            pltpu.async_copy(hs_hbm.at[pl.ds(base + c * 32, 32)],
                             rows_v.at[b], s_in[b][0]),
            pltpu.async_copy(ew128_hbm.at[pl.ds(base + c * 32, 32)],
                             ew_v.at[b], s_in[b][1]),
        )
        for cp in cp_in[b]:
            cp.wait()
        cp_out[b] = (
            pltpu.async_copy(rows_v.at[b], xs_hbm.at[dest2_v.at[c]],
                             s_out[b][0]),
            pltpu.async_copy(ew_v.at[b], ews_hbm.at[dest2_v.at[c]],
                             s_out[b][1]),
        )
    for b in range(2):
        for cp in cp_out[b]:
            cp.wait()


def _moe_body(t_ids, g_ids, nreal, x_ref, w1_ref, b1_ref, w2_ref,
              b2_ref, ew_ref, out_ref):
    i = pl.program_id(0)

    @pl.when(i < nreal[0])
    def _():
        x = x_ref[...].astype(jnp.bfloat16)
        fc1 = jnp.dot(x, w1_ref[0], preferred_element_type=jnp.float32)
        fc1 = fc1 + b1_ref[0]
        act = (0.5 * fc1 * (1.0 + jax.lax.erf(fc1 * 0.7071067811865476))
               ).astype(jnp.bfloat16)
        fc2 = jnp.dot(act, w2_ref[0], preferred_element_type=jnp.float32)
        fc2 = fc2 + b2_ref[0]
        out_ref[...] = fc2 * ew_ref[...][:, :1]


def _grouped_ffn(t_ids, g_ids, nreal, xs, w1, b1, w2, b2, ews):
    grid_spec = pltpu.PrefetchScalarGridSpec(
        num_scalar_prefetch=3,
        grid=(NP,),
        in_specs=[
            pl.BlockSpec((TM, HIDDEN), lambda i, T_, G, N: (T_[i], 0)),
            pl.BlockSpec((1, HIDDEN, FFN), lambda i, T_, G, N: (G[i], 0, 0)),
            pl.BlockSpec((1, 1, FFN), lambda i, T_, G, N: (G[i], 0, 0)),
            pl.BlockSpec((1, FFN, HIDDEN), lambda i, T_, G, N: (G[i], 0, 0)),
            pl.BlockSpec((1, 1, HIDDEN), lambda i, T_, G, N: (G[i], 0, 0)),
            pl.BlockSpec((TM, 128), lambda i, T_, G, N: (T_[i], 0)),
        ],
        out_specs=pl.BlockSpec((TM, HIDDEN), lambda i, T_, G, N: (T_[i], 0)),
    )
    return pl.pallas_call(
        _moe_body,
        grid_spec=grid_spec,
        out_shape=jax.ShapeDtypeStruct((P, HIDDEN), jnp.float32),
        compiler_params=pltpu.CompilerParams(
            dimension_semantics=("arbitrary",)),
    )(t_ids, g_ids, nreal, xs, w1, b1, w2, b2, ews)


def _tile_metadata(counts):
    tiles_g = (counts + TM - 1) // TM
    tile_start = jnp.concatenate(
        [jnp.zeros((1,), jnp.int32),
         jnp.cumsum(tiles_g)[:-1].astype(jnp.int32)])
    ntiles = jnp.sum(tiles_g).astype(jnp.int32)
    padded_offs = tile_start * TM
    idx = jnp.arange(NP, dtype=jnp.int32)
    ii = jnp.minimum(idx, ntiles - 1)
    gid = (jnp.searchsorted(tile_start, ii, side="right").astype(jnp.int32)
           - 1)
    gid = jnp.clip(gid, 0, E - 1)
    return ii, gid, ntiles[None], padded_offs


def kernel(hidden_states, expert_weights, w1, b1, w2, b2, top_experts):
    hidden_shape = hidden_states.shape
    hs = hidden_states.reshape(-1, HIDDEN)
    top = top_experts.reshape(-1).astype(jnp.int32)
    ew = expert_weights.reshape(-1)

    hist = _sc_histogram(top)
    counts = jnp.sum(hist, axis=0).astype(jnp.int32)
    tid, gid, nreal, padded_offs = _tile_metadata(counts)
    start = (padded_offs[None, :]
             + jnp.cumsum(hist, axis=0).astype(jnp.int32) - hist)

    ew128 = jnp.broadcast_to(ew[:, None], (T, 128))
    xs, ews, inv2d = _sc_place_scatter(top, start, hs, ew128)

    out_sorted = _grouped_ffn(tid, gid, nreal, xs,
                              w1.astype(jnp.bfloat16), b1[:, None, :],
                              w2.astype(jnp.bfloat16), b2[:, None, :], ews)

    out = _make_row_gather(T, HIDDEN, jnp.float32, 32)(
        out_sorted, inv2d)
    return out.reshape(hidden_shape)


# speedup vs baseline: 1.0560x; 1.0560x over previous
import functools

import jax
import jax.numpy as jnp
from jax import lax
from jax.experimental import pallas as pl
from jax.experimental.pallas import tpu as pltpu
from jax.experimental.pallas import tpu_sc as plsc

E = 64
HIDDEN = 1024
FFN = 2048
T = 8192
TM = 256
NP = T // TM + E
P = NP * TM


_NW = 32


def _make_row_gather(B, D, dtype, CH):
    bpw = B // _NW
    nch = bpw // CH
    mesh = plsc.VectorSubcoreMesh(core_axis_name="c", subcore_axis_name="s")

    @functools.partial(
        pl.kernel, mesh=mesh,
        out_type=jax.ShapeDtypeStruct((B, D), dtype),
        scratch_types=[
            pltpu.VMEM((nch, CH), jnp.int32),
            pltpu.VMEM((2, CH, D), dtype),
            pltpu.SemaphoreType.DMA,
            pltpu.SemaphoreType.DMA,
        ],
    )
    def gather_k(table_hbm, idx_hbm, out_hbm, idx_v, rows_v, sem0, sem1):
        wid = lax.axis_index("s") * 2 + lax.axis_index("c")
        base = wid * bpw
        pltpu.sync_copy(idx_hbm.at[pl.ds(wid * nch, nch)], idx_v)
        sems = [sem0, sem1]
        cps = [None, None]
        for c in range(nch):
            b = c % 2
            cps[b] = pltpu.async_copy(
                table_hbm.at[idx_v.at[c]], rows_v.at[b], sems[b])
            if c > 0:
                pb = (c - 1) % 2
                cps[pb].wait()
                pltpu.sync_copy(rows_v.at[pb],
                                out_hbm.at[pl.ds(base + (c - 1) * CH, CH)])
        lb = (nch - 1) % 2
        cps[lb].wait()
        pltpu.sync_copy(rows_v.at[lb],
                        out_hbm.at[pl.ds(base + (nch - 1) * CH, CH)])

    return gather_k


_CHUNK = T // _NW

_SC_MESH = dict(core_axis_name="c", subcore_axis_name="s")


def _sc_wid():
    return lax.axis_index("s") * 2 + lax.axis_index("c")


@functools.partial(
    pl.kernel, mesh=plsc.VectorSubcoreMesh(**_SC_MESH),
    out_type=jax.ShapeDtypeStruct((_NW, E), jnp.int32),
    scratch_types=[
        pltpu.VMEM((_CHUNK,), jnp.int32),
        pltpu.VMEM((E,), jnp.int32),
        pltpu.SMEM((E,), jnp.int32),
    ],
)
def _sc_histogram(top_hbm, hist_hbm, keys_v, hist_v, cur_s):
    wid = _sc_wid()
    pltpu.sync_copy(top_hbm.at[pl.ds(wid * _CHUNK, _CHUNK)], keys_v)
    for e in range(E):
        cur_s[e] = 0
    for v in range(_CHUNK // 16):
        kvec = keys_v[pl.ds(v * 16, 16)]
        for l in range(16):
            e = kvec[l]
            cur_s[e] = cur_s[e] + 1
    lanes = lax.broadcasted_iota(jnp.int32, (16,), 0)
    for g in range(E // 16):
        vec = jnp.zeros((16,), jnp.int32)
        for l in range(16):
            vec = jnp.where(lanes == l, cur_s[g * 16 + l], vec)
        hist_v[pl.ds(g * 16, 16)] = vec
    pltpu.sync_copy(hist_v, hist_hbm.at[wid])


@functools.partial(
    pl.kernel, mesh=plsc.VectorSubcoreMesh(**_SC_MESH),
    out_type=(
        jax.ShapeDtypeStruct((P, HIDDEN), jnp.float32),
        jax.ShapeDtypeStruct((P, 128), jnp.float32),
        jax.ShapeDtypeStruct((T // 32, 32), jnp.int32),
    ),
    scratch_types=[
        pltpu.VMEM((_CHUNK,), jnp.int32),
        pltpu.VMEM((8, 32), jnp.int32),
        pltpu.VMEM((2, 32, HIDDEN), jnp.float32),
        pltpu.VMEM((2, 32, 128), jnp.float32),
        pltpu.VMEM((E,), jnp.int32),
        pltpu.SMEM((E,), jnp.int32),
        pltpu.SemaphoreType.DMA,
        pltpu.SemaphoreType.DMA,
        pltpu.SemaphoreType.DMA,
        pltpu.SemaphoreType.DMA,
        pltpu.SemaphoreType.DMA,
        pltpu.SemaphoreType.DMA,
        pltpu.SemaphoreType.DMA,
        pltpu.SemaphoreType.DMA,
    ],
)
def _sc_place_scatter(top_hbm, start_hbm, hs_hbm, ew128_hbm,
                      xs_hbm, ews_hbm, inv_hbm,
                      keys_v, dest2_v, rows_v, ew_v, start_v, cur_s,
                      si0, si1, so0, so1, ei0, ei1, eo0, eo1):
    wid = _sc_wid()
    base = wid * _CHUNK
    pltpu.sync_copy(top_hbm.at[pl.ds(base, _CHUNK)], keys_v)
    pltpu.sync_copy(start_hbm.at[wid], start_v)
    for g in range(E // 16):
        svec = start_v[pl.ds(g * 16, 16)]
        for l in range(16):
            cur_s[g * 16 + l] = svec[l]
    lanes = lax.broadcasted_iota(jnp.int32, (16,), 0)
    for v in range(_CHUNK // 16):
        kvec = keys_v[pl.ds(v * 16, 16)]
        vec = jnp.zeros((16,), jnp.int32)
        for l in range(16):
            e = kvec[l]
            p = cur_s[e]
            cur_s[e] = p + 1
            vec = jnp.where(lanes == l, p, vec)
        dest2_v[v // 2, pl.ds((v % 2) * 16, 16)] = vec
    pltpu.sync_copy(dest2_v, inv_hbm.at[pl.ds(wid * 8, 8)])
    s_in = [(si0, ei0), (si1, ei1)]
    s_out = [(so0, eo0), (so1, eo1)]
    cp_in = [None, None]
    cp_out = [None, None]
    for c in range(8):
        b = c % 2
        if cp_out[b] is not None:
            for cp in cp_out[b]:
                cp.wait()
        cp_in[b] = (
            pltpu.async_copy(hs_hbm.at[pl.ds(base + c * 32, 32)],
                             rows_v.at[b], s_in[b][0]),
            pltpu.async_copy(ew128_hbm.at[pl.ds(base + c * 32, 32)],
                             ew_v.at[b], s_in[b][1]),
        )
        for cp in cp_in[b]:
            cp.wait()
        cp_out[b] = (
            pltpu.async_copy(rows_v.at[b], xs_hbm.at[dest2_v.at[c]],
                             s_out[b][0]),
            pltpu.async_copy(ew_v.at[b], ews_hbm.at[dest2_v.at[c]],
                             s_out[b][1]),
        )
    for b in range(2):
        for cp in cp_out[b]:
            cp.wait()


_DEPTH = 4


def _moe_body(t_ids, g_ids, nreal, run_id, run_gid, nruns,
              x_ref, b1_ref, b2_ref, ew_ref, w1_hbm, w2_hbm, out_ref,
              w1b, w2b, w1s, w2s):
    i = pl.program_id(0)
    r = run_id[i]
    slot = lax.rem(r, _DEPTH)
    prev_r = run_id[jnp.maximum(i - 1, 0)]
    first = jnp.logical_or(i == 0, prev_r != r)

    def issue(rr):
        g = run_gid[rr]
        s = lax.rem(rr, _DEPTH)
        pltpu.make_async_copy(w1_hbm.at[g], w1b.at[s], w1s.at[s]).start()
        pltpu.make_async_copy(w2_hbm.at[g], w2b.at[s], w2s.at[s]).start()

    for k in range(_DEPTH):
        @pl.when(jnp.logical_and(i == 0, nruns[0] > k))
        def _(k=k):
            issue(k)

    @pl.when(jnp.logical_and(first,
                             jnp.logical_and(i > 0,
                                             r + _DEPTH - 1 < nruns[0])))
    def _():
        issue(r + _DEPTH - 1)

    @pl.when(first)
    def _():
        g = run_gid[r]
        pltpu.make_async_copy(w1_hbm.at[g], w1b.at[slot],
                              w1s.at[slot]).wait()
        pltpu.make_async_copy(w2_hbm.at[g], w2b.at[slot],
                              w2s.at[slot]).wait()

    @pl.when(i < nreal[0])
    def _():
        x = x_ref[...].astype(jnp.bfloat16)
        fc1 = jnp.dot(x, w1b[slot], preferred_element_type=jnp.float32)
        fc1 = fc1 + b1_ref[0]
        act = (0.5 * fc1 * (1.0 + jax.lax.erf(fc1 * 0.7071067811865476))
               ).astype(jnp.bfloat16)
        fc2 = jnp.dot(act, w2b[slot], preferred_element_type=jnp.float32)
        fc2 = fc2 + b2_ref[0]
        out_ref[...] = fc2 * ew_ref[...][:, :1]


def _grouped_ffn(t_ids, g_ids, nreal, run_id, run_gid, nruns,
                 xs, w1, b1, w2, b2, ews):
    grid_spec = pltpu.PrefetchScalarGridSpec(
        num_scalar_prefetch=6,
        grid=(NP,),
        in_specs=[
            pl.BlockSpec((TM, HIDDEN),
                         lambda i, T_, G, N, R, RG, NR: (T_[i], 0)),
            pl.BlockSpec((1, 1, FFN),
                         lambda i, T_, G, N, R, RG, NR: (G[i], 0, 0)),
            pl.BlockSpec((1, 1, HIDDEN),
                         lambda i, T_, G, N, R, RG, NR: (G[i], 0, 0)),
            pl.BlockSpec((TM, 128),
                         lambda i, T_, G, N, R, RG, NR: (T_[i], 0)),
            pl.BlockSpec(memory_space=pl.ANY),
            pl.BlockSpec(memory_space=pl.ANY),
        ],
        out_specs=pl.BlockSpec((TM, HIDDEN),
                               lambda i, T_, G, N, R, RG, NR: (T_[i], 0)),
        scratch_shapes=[
            pltpu.VMEM((_DEPTH, HIDDEN, FFN), jnp.bfloat16),
            pltpu.VMEM((_DEPTH, FFN, HIDDEN), jnp.bfloat16),
            pltpu.SemaphoreType.DMA((_DEPTH,)),
            pltpu.SemaphoreType.DMA((_DEPTH,)),
        ],
    )
    return pl.pallas_call(
        _moe_body,
        grid_spec=grid_spec,
        out_shape=jax.ShapeDtypeStruct((P, HIDDEN), jnp.float32),
        compiler_params=pltpu.CompilerParams(
            dimension_semantics=("arbitrary",)),
    )(t_ids, g_ids, nreal, run_id, run_gid, nruns,
      xs, b1, b2, ews, w1, w2)


def _tile_metadata(counts):
    tiles_g = (counts + TM - 1) // TM
    tile_start = jnp.concatenate(
        [jnp.zeros((1,), jnp.int32),
         jnp.cumsum(tiles_g)[:-1].astype(jnp.int32)])
    ntiles = jnp.sum(tiles_g).astype(jnp.int32)
    padded_offs = tile_start * TM
    idx = jnp.arange(NP, dtype=jnp.int32)
    ii = jnp.minimum(idx, ntiles - 1)
    gid = (jnp.searchsorted(tile_start, ii, side="right").astype(jnp.int32)
           - 1)
    gid = jnp.clip(gid, 0, E - 1)
    valid = idx < ntiles
    newrun = jnp.logical_and(
        valid,
        jnp.concatenate([jnp.ones((1,), jnp.bool_), gid[1:] != gid[:-1]]))
    run_id = jnp.cumsum(newrun.astype(jnp.int32)) - 1
    nruns = run_id[ntiles - 1] + 1
    run_gid = jnp.zeros((NP,), jnp.int32).at[run_id].set(gid)
    return ii, gid, ntiles[None], padded_offs, run_id, nruns[None], run_gid


def kernel(hidden_states, expert_weights, w1, b1, w2, b2, top_experts):
    hidden_shape = hidden_states.shape
    hs = hidden_states.reshape(-1, HIDDEN)
    top = top_experts.reshape(-1).astype(jnp.int32)
    ew = expert_weights.reshape(-1)

    hist = _sc_histogram(top)
    counts = jnp.sum(hist, axis=0).astype(jnp.int32)
    (tid, gid, nreal, padded_offs,
     run_id, nruns, run_gid) = _tile_metadata(counts)
    start = (padded_offs[None, :]
             + jnp.cumsum(hist, axis=0).astype(jnp.int32) - hist)

    ew128 = jnp.broadcast_to(ew[:, None], (T, 128))
    xs, ews, inv2d = _sc_place_scatter(top, start, hs, ew128)

    out_sorted = _grouped_ffn(tid, gid, nreal, run_id, run_gid, nruns, xs,
                              w1.astype(jnp.bfloat16), b1[:, None, :],
                              w2.astype(jnp.bfloat16), b2[:, None, :], ews)

    out = _make_row_gather(T, HIDDEN, jnp.float32, 32)(
        out_sorted, inv2d)
    return out.reshape(hidden_shape)
